# baseline (speedup 1.0000x reference)
import functools

import jax
import jax.numpy as jnp
from jax import lax
from jax.experimental import pallas as pl
from jax.experimental.pallas import tpu as pltpu
from jax.experimental.pallas import tpu_sc as plsc

B, N, K = 8, 2048, 16
RB = 256
NB = N // RB
NW = 32
PPW = (B * N) // NW
PCH = 8
NCH = PPW // PCH


def _knn_body(cb_ref, ct_ref, out_ref):
    b = pl.program_id(0)
    rb = pl.program_id(1)
    cb = cb_ref[0]
    ct = ct_ref[0]
    inner = jnp.dot(cb.astype(jnp.bfloat16), ct.astype(jnp.bfloat16),
                    preferred_element_type=jnp.float32)
    nr = jnp.sum(cb * cb, axis=1, keepdims=True)
    nc = jnp.sum(ct * ct, axis=0, keepdims=True)
    d = nr - 2.0 * inner + nc
    col = lax.broadcasted_iota(jnp.int32, d.shape, 1)
    rowg = rb * RB + lax.broadcasted_iota(jnp.int32, d.shape, 0)
    d = d + jnp.where(col == rowg, jnp.float32(1e9), jnp.float32(0.0))
    big = jnp.float32(3e9)
    cols = []
    for _ in range(K):
        m = jnp.min(d, axis=1, keepdims=True)
        cand = jnp.where(d <= m, col, jnp.int32(1 << 30))
        j = jnp.min(cand, axis=1, keepdims=True)
        cols.append(j)
        d = jnp.where(col == j, big, d)
    idx = jnp.concatenate(cols, axis=1)
    out_ref[0] = idx + lax.rem(b, 2) * N


def _knn(cp, ct):
    return pl.pallas_call(
        _knn_body,
        grid=(B, NB),
        in_specs=[
            pl.BlockSpec((1, RB, 8), lambda b, r: (b, r, 0)),
            pl.BlockSpec((1, 8, N), lambda b, r: (b, 0, 0)),
        ],
        out_specs=pl.BlockSpec((1, RB, K), lambda b, r: (b, r, 0)),
        out_shape=jax.ShapeDtypeStruct((B, N, K), jnp.int32),
    )(cp, ct)


def _chunk_max(buf, out_v, c):
    for p in range(PCH):
        for q in range(4):
            acc = buf[p * K, pl.ds(64 + q * 16, 16)]
            for r in range(1, K):
                acc = jnp.maximum(acc, buf[p * K + r, pl.ds(64 + q * 16, 16)])
            out_v[c * PCH + p, pl.ds(q * 16, 16)] = acc


def _gmax_body(t_hbm, idx_hbm, out_hbm, idx_v, out_v, buf0, buf1, spm,
               sem0, sem1):
    cid = lax.axis_index("c")
    sid = lax.axis_index("s")
    for r in range(2):
        bb = 4 * cid + 2 * r
        row0 = bb * N
        pltpu.sync_copy(t_hbm.at[pl.ds(row0 + sid * 256, 256)],
                        spm.at[pl.ds(sid * 256, 256)])
        blk = bb * 8 + sid
        pltpu.sync_copy(idx_hbm.at[blk], idx_v)
        plsc.subcore_barrier()

        def pair(i, carry):
            c0 = 2 * i
            c1 = 2 * i + 1
            cp0 = pltpu.async_copy(spm.at[idx_v.at[c0]], buf0, sem0)
            cp1 = pltpu.async_copy(spm.at[idx_v.at[c1]], buf1, sem1)
            cp0.wait()
            _chunk_max(buf0, out_v, c0)
            cp1.wait()
            _chunk_max(buf1, out_v, c1)
            return carry

        lax.fori_loop(0, 16, pair, 0)
        pltpu.sync_copy(out_v, out_hbm.at[pl.ds(blk * 256, 256)])
        plsc.subcore_barrier()


@functools.partial(jax.jit, static_argnames=())
def _gmax(t, idx_sc):
    mesh = plsc.VectorSubcoreMesh(core_axis_name="c", subcore_axis_name="s",
                                  num_cores=2)
    kern = functools.partial(
        pl.kernel,
        mesh=mesh,
        out_type=jax.ShapeDtypeStruct((B * N, 64), jnp.float32),
        scratch_types=[
            pltpu.VMEM((32, 128), jnp.int32),
            pltpu.VMEM((256, 64), jnp.float32),
            pltpu.VMEM((PCH * K, 128), jnp.float32),
            pltpu.VMEM((PCH * K, 128), jnp.float32),
            pltpu.VMEM_SHARED((2 * N, 128), jnp.float32),
            pltpu.SemaphoreType.DMA,
            pltpu.SemaphoreType.DMA,
        ],
    )(_gmax_body)
    return kern(t, idx_sc)


def _mm(x, w):
    return jnp.dot(x, w, preferred_element_type=jnp.float32,
                   precision=lax.Precision.HIGHEST)


def _relu(x):
    return jnp.maximum(x, jnp.float32(0.0))


def _fuse0_body(c_ref, f_ref, j_ref, w1_ref, b1_ref, w2_ref, b2_ref,
                wdx_ref, wdf_ref, wdj_ref, be1_ref, wbx_ref, wbf_ref,
                wbj_ref, at_ref):
    x = _relu(_mm(c_ref[0], w1_ref[...]) + b1_ref[...])
    x = _relu(_mm(x, w2_ref[...]) + b2_ref[...])
    f = f_ref[0]
    jt = j_ref[0]
    a = (_mm(x, wdx_ref[...]) + _mm(f, wdf_ref[...])
         + _mm(jt, wdj_ref[...]) + be1_ref[...])
    t = (_mm(x, wbx_ref[...]) + _mm(f, wbf_ref[...])
         + _mm(jt, wbj_ref[...]))
    at_ref[0] = jnp.concatenate([a, t], axis=1)


def _mid1_body(at_ref, m_ref, wd_ref, be_ref, wb_ref, ato_ref):
    x = _relu(at_ref[0][:, :64] + m_ref[0])
    a = _mm(x, wd_ref[...]) + be_ref[...]
    t = _mm(x, wb_ref[...])
    ato_ref[0] = jnp.concatenate([a, t], axis=1)


def _mid2_body(at_ref, m_ref, w3_ref, b3_ref, wd_ref, bc_ref, wb_ref,
               ato_ref):
    x = _relu(at_ref[0][:, :64] + m_ref[0])
    y = _relu(_mm(x, w3_ref[...]) + b3_ref[...])
    a = _mm(y, wd_ref[...]) + bc_ref[...]
    t = _mm(y, wb_ref[...])
    ato_ref[0] = jnp.concatenate([a, t], axis=1)


def _mid3_body(at_ref, m_ref, wd_ref, bc_ref, wb_ref, ato_ref, eo_ref):
    e1 = _relu(at_ref[0][:, :64] + m_ref[0])
    eo_ref[0] = e1
    a = _mm(e1, wd_ref[...]) + bc_ref[...]
    t = _mm(e1, wb_ref[...])
    ato_ref[0] = jnp.concatenate([a, t], axis=1)


def _mid4_body(at_ref, m_ref, e1_ref, wd_ref, bc_ref, wb_ref,
               ato_ref, eo_ref):
    e2 = _relu(at_ref[0][:, :64] + m_ref[0])
    eo_ref[0] = e2
    xin = e2 + e1_ref[0]
    a = _mm(xin, wd_ref[...]) + bc_ref[...]
    t = _mm(xin, wb_ref[...])
    ato_ref[0] = jnp.concatenate([a, t], axis=1)


def _fin_body(at_ref, m_ref, e2_ref, w4_ref, b4_ref, wo1_ref, bo1_ref,
              w5_ref, b5_ref, wo2_ref, bo2_ref, o1_ref, o2_ref):
    x = _relu(at_ref[0][:, :64] + m_ref[0]) + e2_ref[0]
    h1 = _relu(_mm(x, w4_ref[...]) + b4_ref[...])
    o1_ref[0] = _relu(_mm(h1, wo1_ref[...]) + bo1_ref[...])
    h2 = _relu(_mm(x, w5_ref[...]) + b5_ref[...])
    o2_ref[0] = _relu(_mm(h2, wo2_ref[...]) + bo2_ref[...])


def _row_spec(c):
    return pl.BlockSpec((1, N, c), lambda b: (b, 0, 0))


def _w_spec(shape):
    nd = len(shape)
    return pl.BlockSpec(shape, lambda b: (0,) * nd)


def _call(body, ins, row_in_cols, w_shapes, out_cols):
    in_specs = ([_row_spec(c) for c in row_in_cols]
                + [_w_spec(s) for s in w_shapes])
    out_specs = tuple(_row_spec(c) for c in out_cols)
    out_shape = tuple(jax.ShapeDtypeStruct((B, N, c), jnp.float32)
                      for c in out_cols)
    if len(out_cols) == 1:
        out_specs = out_specs[0]
        out_shape = out_shape[0]
    return pl.pallas_call(
        body, grid=(B,), in_specs=in_specs,
        out_specs=out_specs, out_shape=out_shape,
    )(*ins)


def kernel(Coordinate3D, Feature512D, JointType, W1, b1, W2, b2, We1, be1,
           We2, be2, W3, b3, Wc1, bc1, Wc2, bc2, Wc3, bc3, W4, b4, Wo1, bo1,
           W5, b5, Wo2, bo2):
    f32 = jnp.float32
    cp = jnp.pad(Coordinate3D, ((0, 0), (0, 0), (0, 5)))
    ct = jnp.transpose(cp, (0, 2, 1))
    idx = _knn(cp, ct)
    idx_sc = idx.reshape(64, 32, 128)

    def r2(v):
        return v.reshape(1, -1).astype(f32)

    w1p = jnp.pad(W1, ((0, 5), (0, 0)))
    wa, wb = We1[:592], We1[592:]
    wd = wa - wb
    wdx, wdf, wdj = wd[:64], wd[64:576], wd[576:]
    wbx, wbf, wbj = wb[:64], wb[64:576], wb[576:]

    def split(w):
        return w[:64] - w[64:], w[64:]

    wd2, wb2 = split(We2)
    wd3, wb3 = split(Wc1)
    wd4, wb4 = split(Wc2)
    wd5, wb5 = split(Wc3)

    wo1p = jnp.zeros((64, 128), f32).at[:, :3].set(Wo1)
    bo1p = jnp.zeros((1, 128), f32).at[:, :3].set(bo1)
    wo2p = jnp.zeros((64, 128), f32).at[:, :21].set(Wo2)
    bo2p = jnp.zeros((1, 128), f32).at[:, :21].set(bo2)

    at1 = _call(
        _fuse0_body,
        [cp, Feature512D, JointType, w1p, r2(b1), W2, r2(b2),
         wdx, wdf, wdj, r2(be1), wbx, wbf, wbj],
        [8, 512, 16],
        [(8, 64), (1, 64), (64, 64), (1, 64), (64, 64), (512, 64), (16, 64),
         (1, 64), (64, 64), (512, 64), (16, 64)],
        [128])
    m1 = _gmax(at1.reshape(B * N, 128), idx_sc).reshape(B, N, 64)

    at2 = _call(_mid1_body, [at1, m1, wd2, r2(be2), wb2], [128, 64],
                [(64, 64), (1, 64), (64, 64)], [128])
    m2 = _gmax(at2.reshape(B * N, 128), idx_sc).reshape(B, N, 64)

    at3 = _call(_mid2_body, [at2, m2, W3, r2(b3), wd3, r2(bc1), wb3],
                [128, 64],
                [(64, 64), (1, 64), (64, 64), (1, 64), (64, 64)], [128])
    m3 = _gmax(at3.reshape(B * N, 128), idx_sc).reshape(B, N, 64)

    at4, e1 = _call(_mid3_body, [at3, m3, wd4, r2(bc2), wb4], [128, 64],
                    [(64, 64), (1, 64), (64, 64)], [128, 64])
    m4 = _gmax(at4.reshape(B * N, 128), idx_sc).reshape(B, N, 64)

    at5, e2 = _call(_mid4_body, [at4, m4, e1, wd5, r2(bc3), wb5],
                    [128, 64, 64],
                    [(64, 64), (1, 64), (64, 64)], [128, 64])
    m5 = _gmax(at5.reshape(B * N, 128), idx_sc).reshape(B, N, 64)

    o1, o2 = _call(
        _fin_body,
        [at5, m5, e2, W4, r2(b4), wo1p, bo1p, W5, r2(b5), wo2p, bo2p],
        [128, 64, 64],
        [(64, 64), (1, 64), (64, 128), (1, 128), (64, 64), (1, 64),
         (64, 128), (1, 128)],
        [128, 128])
    return (o1[:, :, :3], o2[:, :, :21])


# speedup vs baseline: 1.0381x; 1.0381x over previous
import functools

import jax
import jax.numpy as jnp
from jax import lax
from jax.experimental import pallas as pl
from jax.experimental.pallas import tpu as pltpu
from jax.experimental.pallas import tpu_sc as plsc

B, N, K = 8, 2048, 16
RB = 256
NB = N // RB
NW = 32
PPW = (B * N) // NW
PCH = 8
NCH = PPW // PCH


def _knn_body(cb_ref, ct_ref, out_ref):
    b = pl.program_id(0)
    rb = pl.program_id(1)
    cb = cb_ref[0]
    ct = ct_ref[0]
    inner = jnp.dot(cb.astype(jnp.bfloat16), ct.astype(jnp.bfloat16),
                    preferred_element_type=jnp.float32)
    nr = jnp.sum(cb * cb, axis=1, keepdims=True)
    nc = jnp.sum(ct * ct, axis=0, keepdims=True)
    d = nr - 2.0 * inner + nc
    col = lax.broadcasted_iota(jnp.int32, d.shape, 1)
    rowg = rb * RB + lax.broadcasted_iota(jnp.int32, d.shape, 0)
    d = d + jnp.where(col == rowg, jnp.float32(1e9), jnp.float32(0.0))
    big = jnp.float32(3e9)
    cols = []
    for _ in range(K):
        m = jnp.min(d, axis=1, keepdims=True)
        cand = jnp.where(d <= m, col, jnp.int32(1 << 30))
        j = jnp.min(cand, axis=1, keepdims=True)
        cols.append(j)
        d = jnp.where(col == j, big, d)
    idx = jnp.concatenate(cols, axis=1)
    out_ref[0] = idx + lax.rem(b, 2) * N


def _knn(cp, ct):
    return pl.pallas_call(
        _knn_body,
        grid=(B, NB),
        in_specs=[
            pl.BlockSpec((1, RB, 8), lambda b, r: (b, r, 0)),
            pl.BlockSpec((1, 8, N), lambda b, r: (b, 0, 0)),
        ],
        out_specs=pl.BlockSpec((1, RB, K), lambda b, r: (b, r, 0)),
        out_shape=jax.ShapeDtypeStruct((B, N, K), jnp.int32),
    )(cp, ct)


def _chunk_max(buf, out_v, c):
    for p in range(PCH):
        for q in range(4):
            sl = pl.ds(64 + q * 16, 16)
            v = [buf[p * K + r, sl] for r in range(K)]
            while len(v) > 1:
                v = [jnp.maximum(v[i], v[i + 1]) for i in range(0, len(v), 2)]
            out_v[c * PCH + p, pl.ds(q * 16, 16)] = v[0]


def _gmax_body(t_hbm, idx_hbm, out_hbm, idx_v, out_v, buf0, buf1, spm,
               sem0, sem1):
    cid = lax.axis_index("c")
    sid = lax.axis_index("s")
    for r in range(2):
        bb = 4 * cid + 2 * r
        row0 = bb * N
        pltpu.sync_copy(t_hbm.at[pl.ds(row0 + sid * 256, 256)],
                        spm.at[pl.ds(sid * 256, 256)])
        blk = bb * 8 + sid
        pltpu.sync_copy(idx_hbm.at[blk], idx_v)
        plsc.subcore_barrier()

        def pair(i, carry):
            c0 = 2 * i
            c1 = 2 * i + 1
            cp0 = pltpu.async_copy(spm.at[idx_v.at[c0]], buf0, sem0)
            cp1 = pltpu.async_copy(spm.at[idx_v.at[c1]], buf1, sem1)
            cp0.wait()
            _chunk_max(buf0, out_v, c0)
            cp1.wait()
            _chunk_max(buf1, out_v, c1)
            return carry

        lax.fori_loop(0, 16, pair, 0)
        pltpu.sync_copy(out_v, out_hbm.at[pl.ds(blk * 256, 256)])
        plsc.subcore_barrier()


@functools.partial(jax.jit, static_argnames=())
def _gmax(t, idx_sc):
    mesh = plsc.VectorSubcoreMesh(core_axis_name="c", subcore_axis_name="s",
                                  num_cores=2)
    kern = functools.partial(
        pl.kernel,
        mesh=mesh,
        out_type=jax.ShapeDtypeStruct((B * N, 64), jnp.float32),
        scratch_types=[
            pltpu.VMEM((32, 128), jnp.int32),
            pltpu.VMEM((256, 64), jnp.float32),
            pltpu.VMEM((PCH * K, 128), jnp.float32),
            pltpu.VMEM((PCH * K, 128), jnp.float32),
            pltpu.VMEM_SHARED((2 * N, 128), jnp.float32),
            pltpu.SemaphoreType.DMA,
            pltpu.SemaphoreType.DMA,
        ],
    )(_gmax_body)
    return kern(t, idx_sc)


def _mm(x, w):
    return jnp.dot(x.astype(jnp.bfloat16), w,
                   preferred_element_type=jnp.float32)


def _mmh(x, w):
    return jnp.dot(x, w, preferred_element_type=jnp.float32,
                   precision=lax.Precision.HIGHEST)


def _relu(x):
    return jnp.maximum(x, jnp.float32(0.0))


def _fuse0_body(c_ref, f_ref, j_ref, w1_ref, b1_ref, w2_ref, b2_ref,
                wdx_ref, wdf_ref, wdj_ref, be1_ref, wbx_ref, wbf_ref,
                wbj_ref, at_ref):
    x = _relu(_mm(c_ref[0], w1_ref[...]) + b1_ref[...])
    x = _relu(_mm(x, w2_ref[...]) + b2_ref[...])
    f = f_ref[0]
    jt = j_ref[0]
    t = (_mmh(x, wbx_ref[...]) + _mmh(f, wbf_ref[...])
         + _mmh(jt, wbj_ref[...]))
    a = (_mm(x, wdx_ref[...]) + _mm(f, wdf_ref[...])
         + _mm(jt, wdj_ref[...]) + be1_ref[...]) - t
    at_ref[0] = jnp.concatenate([a, t], axis=1)


def _mid1_body(at_ref, m_ref, wd_ref, be_ref, wb_ref, ato_ref):
    x = _relu(at_ref[0][:, :64] + m_ref[0])
    t = _mmh(x, wb_ref[...])
    a = _mm(x, wd_ref[...]) + be_ref[...] - t
    ato_ref[0] = jnp.concatenate([a, t], axis=1)


def _mid2_body(at_ref, m_ref, w3_ref, b3_ref, wd_ref, bc_ref, wb_ref,
               ato_ref):
    x = _relu(at_ref[0][:, :64] + m_ref[0])
    y = _relu(_mm(x, w3_ref[...]) + b3_ref[...])
    t = _mmh(y, wb_ref[...])
    a = _mm(y, wd_ref[...]) + bc_ref[...] - t
    ato_ref[0] = jnp.concatenate([a, t], axis=1)


def _mid3_body(at_ref, m_ref, wd_ref, bc_ref, wb_ref, ato_ref, eo_ref):
    e1 = _relu(at_ref[0][:, :64] + m_ref[0])
    eo_ref[0] = e1
    t = _mmh(e1, wb_ref[...])
    a = _mm(e1, wd_ref[...]) + bc_ref[...] - t
    ato_ref[0] = jnp.concatenate([a, t], axis=1)


def _mid4_body(at_ref, m_ref, e1_ref, wd_ref, bc_ref, wb_ref,
               ato_ref, eo_ref):
    e2 = _relu(at_ref[0][:, :64] + m_ref[0])
    eo_ref[0] = e2
    xin = e2 + e1_ref[0]
    t = _mmh(xin, wb_ref[...])
    a = _mm(xin, wd_ref[...]) + bc_ref[...] - t
    ato_ref[0] = jnp.concatenate([a, t], axis=1)


def _fin_body(at_ref, m_ref, e2_ref, w4_ref, b4_ref, wo1_ref, bo1_ref,
              w5_ref, b5_ref, wo2_ref, bo2_ref, o1_ref, o2_ref):
    x = _relu(at_ref[0][:, :64] + m_ref[0]) + e2_ref[0]
    h1 = _relu(_mm(x, w4_ref[...]) + b4_ref[...])
    o1_ref[0] = _relu(_mm(h1, wo1_ref[...]) + bo1_ref[...])
    h2 = _relu(_mm(x, w5_ref[...]) + b5_ref[...])
    o2_ref[0] = _relu(_mm(h2, wo2_ref[...]) + bo2_ref[...])


def _row_spec(c):
    return pl.BlockSpec((1, N, c), lambda b: (b, 0, 0))


def _w_spec(shape):
    nd = len(shape)
    return pl.BlockSpec(shape, lambda b: (0,) * nd)


def _call(body, ins, row_in_cols, w_shapes, out_cols):
    in_specs = ([_row_spec(c) for c in row_in_cols]
                + [_w_spec(s) for s in w_shapes])
    out_specs = tuple(_row_spec(c) for c in out_cols)
    out_shape = tuple(jax.ShapeDtypeStruct((B, N, c), jnp.float32)
                      for c in out_cols)
    if len(out_cols) == 1:
        out_specs = out_specs[0]
        out_shape = out_shape[0]
    return pl.pallas_call(
        body, grid=(B,), in_specs=in_specs,
        out_specs=out_specs, out_shape=out_shape,
    )(*ins)


def kernel(Coordinate3D, Feature512D, JointType, W1, b1, W2, b2, We1, be1,
           We2, be2, W3, b3, Wc1, bc1, Wc2, bc2, Wc3, bc3, W4, b4, Wo1, bo1,
           W5, b5, Wo2, bo2):
    f32 = jnp.float32
    cp = jnp.pad(Coordinate3D, ((0, 0), (0, 0), (0, 5)))
    ct = jnp.transpose(cp, (0, 2, 1))
    idx = _knn(cp, ct)
    idx_sc = idx.reshape(64, 32, 128)

    def r2(v):
        return v.reshape(1, -1).astype(f32)

    bf = jnp.bfloat16
    w1p = jnp.pad(W1, ((0, 5), (0, 0))).astype(bf)
    wa = We1[:592].astype(bf)
    wb = We1[592:].astype(bf).astype(f32)
    wdx, wdf, wdj = wa[:64], wa[64:576], wa[576:]
    wbx, wbf, wbj = wb[:64], wb[64:576], wb[576:]

    def split(w):
        return w[:64].astype(bf), w[64:].astype(bf).astype(f32)

    wd2, wb2 = split(We2)
    wd3, wb3 = split(Wc1)
    wd4, wb4 = split(Wc2)
    wd5, wb5 = split(Wc3)

    wo1p = jnp.zeros((64, 128), f32).at[:, :3].set(Wo1).astype(bf)
    bo1p = jnp.zeros((1, 128), f32).at[:, :3].set(bo1)
    wo2p = jnp.zeros((64, 128), f32).at[:, :21].set(Wo2).astype(bf)
    bo2p = jnp.zeros((1, 128), f32).at[:, :21].set(bo2)

    at1 = _call(
        _fuse0_body,
        [cp, Feature512D, JointType, w1p, r2(b1), W2.astype(bf), r2(b2),
         wdx, wdf, wdj, r2(be1), wbx, wbf, wbj],
        [8, 512, 16],
        [(8, 64), (1, 64), (64, 64), (1, 64), (64, 64), (512, 64), (16, 64),
         (1, 64), (64, 64), (512, 64), (16, 64)],
        [128])
    m1 = _gmax(at1.reshape(B * N, 128), idx_sc).reshape(B, N, 64)

    at2 = _call(_mid1_body, [at1, m1, wd2, r2(be2), wb2], [128, 64],
                [(64, 64), (1, 64), (64, 64)], [128])
    m2 = _gmax(at2.reshape(B * N, 128), idx_sc).reshape(B, N, 64)

    at3 = _call(_mid2_body, [at2, m2, W3.astype(bf), r2(b3), wd3, r2(bc1), wb3],
                [128, 64],
                [(64, 64), (1, 64), (64, 64), (1, 64), (64, 64)], [128])
    m3 = _gmax(at3.reshape(B * N, 128), idx_sc).reshape(B, N, 64)

    at4, e1 = _call(_mid3_body, [at3, m3, wd4, r2(bc2), wb4], [128, 64],
                    [(64, 64), (1, 64), (64, 64)], [128, 64])
    m4 = _gmax(at4.reshape(B * N, 128), idx_sc).reshape(B, N, 64)

    at5, e2 = _call(_mid4_body, [at4, m4, e1, wd5, r2(bc3), wb5],
                    [128, 64, 64],
                    [(64, 64), (1, 64), (64, 64)], [128, 64])
    m5 = _gmax(at5.reshape(B * N, 128), idx_sc).reshape(B, N, 64)

    o1, o2 = _call(
        _fin_body,
        [at5, m5, e2, W4.astype(bf), r2(b4), wo1p, bo1p, W5.astype(bf),
         r2(b5), wo2p, bo2p],
        [128, 64, 64],
        [(64, 64), (1, 64), (64, 128), (1, 128), (64, 64), (1, 64),
         (64, 128), (1, 128)],
        [128, 128])
    return (o1[:, :, :3], o2[:, :, :21])
